# baseline (speedup 1.0000x reference)
import functools

import jax
import jax.numpy as jnp
from jax import lax
from jax.experimental import pallas as pl
from jax.experimental.pallas import tpu as pltpu
from jax.experimental.pallas import tpu_sc as plsc

N = 50000
E = 800000
D = 96
NC = 2
NS = 16
CPT = 200
NCHUNK = 32 * CPT
EPAD = NCHUNK * 128
DUMMY = 50040
NPAD = 50048
ZPT = NPAD // NS
EPT = EPAD // 32
MG = 5


@functools.cache
def _mesh():
    return plsc.VectorSubcoreMesh(core_axis_name="c", subcore_axis_name="s",
                                  num_cores=NC, num_subcores=NS)


CHG = 400
NCHG = EPT // CHG


def _sc_gather2(table, src1d, dst1d):
    return _make_gather2()(table, src1d, dst1d)


def _gather2_body(table, src1d, dst1d, hs_out, hd_out,
                  idx0, idx1, rows0, rows1, gsem0, gsem1, wsem0, wsem1):
    c = lax.axis_index("c")
    s = lax.axis_index("s")
    wid = c * NS + s
    base0 = wid * EPT
    BUFS = ((idx0, rows0, gsem0, wsem0), (idx1, rows1, gsem1, wsem1))

    def phase(idx1d_hbm, out_hbm):
        def drain(buf, sem):
            pltpu.make_async_copy(out_hbm.at[pl.ds(0, CHG)], buf, sem).wait()

        def stage(i, idxb, rowsb, gsem):
            pltpu.sync_copy(idx1d_hbm.at[pl.ds(base0 + i * CHG, CHG)], idxb)
            pltpu.async_copy(table.at[idxb], rowsb, gsem)

        stage(0, *BUFS[0][:3])

        def pair(j, carry):
            for b in range(2):
                i = j * 2 + b
                idxb, rowsb, gsem, wsem = BUFS[b]
                nidxb, nrowsb, ngsem, nwsem = BUFS[1 - b]

                @pl.when(i + 1 < NCHG)
                def _():
                    @pl.when(i >= 1)
                    def _():
                        drain(nrowsb, nwsem)
                    stage(i + 1, nidxb, nrowsb, ngsem)

                drain(rowsb, gsem)
                pltpu.async_copy(rowsb,
                                 out_hbm.at[pl.ds(base0 + i * CHG, CHG)],
                                 wsem)
            return carry

        lax.fori_loop(0, NCHG // 2, pair, 0)
        drain(rows0, wsem0)
        drain(rows1, wsem1)

    phase(src1d, hs_out)
    phase(dst1d, hd_out)


@functools.cache
def _make_gather2():
    return functools.partial(
        pl.kernel,
        out_type=(jax.ShapeDtypeStruct((EPAD, 128), jnp.float32),
                  jax.ShapeDtypeStruct((EPAD, 128), jnp.float32)),
        mesh=_mesh(),
        scratch_types=[
            pltpu.VMEM((CHG,), jnp.int32),
            pltpu.VMEM((CHG,), jnp.int32),
            pltpu.VMEM((CHG, 128), jnp.float32),
            pltpu.VMEM((CHG, 128), jnp.float32),
            pltpu.SemaphoreType.DMA,
            pltpu.SemaphoreType.DMA,
            pltpu.SemaphoreType.DMA,
            pltpu.SemaphoreType.DMA,
        ],
        compiler_params=pltpu.CompilerParams(use_tc_tiling_on_sc=False),
    )(_gather2_body)


def _sc_scatter(m0, m1, m2, dst1d, zeros_hbm):
    return _make_scatter()(m0, m1, m2, dst1d, zeros_hbm)


def _scatter_body(m0, m1, m2, dst1d, zeros_hbm, zp_out, didx, mbuf, zacc, sem):
    c = lax.axis_index("c")
    s = lax.axis_index("s")
    edge0 = (c * NS + s) * EPT

    for p, m_hbm in enumerate((m0, m1, m2)):
        pltpu.sync_copy(zeros_hbm, zacc.at[pl.ds(s * ZPT, ZPT)])
        plsc.subcore_barrier()

        def group(g, carry):
            b = edge0 + g * (MG * 128)
            pltpu.sync_copy(dst1d.at[pl.ds(b, MG * 128)], didx)
            pltpu.sync_copy(m_hbm.at[pl.ds(b, MG * 128)], mbuf)
            pltpu.sync_copy(mbuf, zacc.at[didx], add=True)
            return carry

        lax.fori_loop(0, EPT // (MG * 128), group, 0)
        plsc.subcore_barrier()

        @pl.when(s < NS - 1)
        def _():
            pltpu.sync_copy(zacc.at[pl.ds(s * ZPT, ZPT)],
                            zp_out.at[c, p, pl.ds(s * ZPT, ZPT)])

        @pl.when(s == NS - 1)
        def _():
            pltpu.sync_copy(
                zacc.at[pl.ds((NS - 1) * ZPT, N - (NS - 1) * ZPT)],
                zp_out.at[c, p, pl.ds((NS - 1) * ZPT, N - (NS - 1) * ZPT)])

        plsc.subcore_barrier()


@functools.cache
def _make_scatter():
    return functools.partial(
        pl.kernel,
        out_type=jax.ShapeDtypeStruct((2, 3, N, 32), jnp.float32),
        mesh=_mesh(),
        scratch_types=[
            pltpu.VMEM((MG * 128,), jnp.int32),
            pltpu.VMEM((MG * 128, 32), jnp.float32),
            pltpu.VMEM_SHARED((NPAD, 32), jnp.float32),
            pltpu.SemaphoreType.DMA,
        ],
        compiler_params=pltpu.CompilerParams(use_tc_tiling_on_sc=False),
    )(_scatter_body)


def _prelu(x, a):
    return jnp.where(x >= 0, x, a * x)


def _prologue_call(hp, hm, wp1, bp1, wp2, bp2, wm1, bm1, wm2, bm2, ap, am):
    R, BLK = 25000, 5000

    def body(hp_ref, hm_ref, wp1_ref, bp1_ref, wp2_ref, bp2_ref,
             wm1_ref, bm1_ref, wm2_ref, bm2_ref, ap_ref, am_ref,
             xp_ref, xm_ref):
        t = jnp.dot(hp_ref[...], wp1_ref[...],
                    preferred_element_type=jnp.float32) + bp1_ref[...]
        t = _prelu(t, ap_ref[0])
        xp_ref[...] = jnp.dot(t, wp2_ref[...],
                              preferred_element_type=jnp.float32) + bp2_ref[...]
        u = jnp.dot(hm_ref[...], wm1_ref[...],
                    preferred_element_type=jnp.float32) + bm1_ref[...]
        u = _prelu(u, am_ref[0])
        xm_ref[...] = jnp.dot(u, wm2_ref[...],
                              preferred_element_type=jnp.float32) + bm2_ref[...]

    full = lambda shape: pl.BlockSpec(shape, lambda i: (0, 0))
    smem = pl.BlockSpec(memory_space=pltpu.MemorySpace.SMEM)
    return pl.pallas_call(
        body,
        grid=(R // BLK,),
        in_specs=[
            pl.BlockSpec((BLK, 192), lambda i: (i, 0)),
            pl.BlockSpec((BLK, 192), lambda i: (i, 0)),
            full((192, 192)), full((1, 192)), full((192, D)), full((1, D)),
            full((192, 192)), full((1, 192)), full((192, D)), full((1, D)),
            smem, smem,
        ],
        out_specs=[pl.BlockSpec((BLK, D), lambda i: (i, 0)),
                   pl.BlockSpec((BLK, D), lambda i: (i, 0))],
        out_shape=[jax.ShapeDtypeStruct((R, D), jnp.float32),
                   jax.ShapeDtypeStruct((R, D), jnp.float32)],
    )(hp, hm, wp1, bp1, wp2, bp2, wm1, bm1, wm2, bm2, ap, am)


def _edge_call(hs, hd, heads):
    BLK = 4096

    def body(*refs):
        hs_ref, hd_ref = refs[0], refs[1]
        wrefs = refs[2:2 + 3 * len(heads)]
        srefs = refs[2 + 3 * len(heads):2 + 5 * len(heads)]
        orefs = refs[2 + 5 * len(heads):]
        hsv = hs_ref[...]
        h2 = hsv * hd_ref[...]
        o = 0
        for i, head in enumerate(heads):
            kind = head[0]
            w1t_ref, b1_ref, w2_ref = wrefs[3 * i], wrefs[3 * i + 1], wrefs[3 * i + 2]
            a_ref, b2_ref = srefs[2 * i], srefs[2 * i + 1]
            u = jnp.dot(h2, w1t_ref[...],
                        preferred_element_type=jnp.float32) + b1_ref[...]
            g = _prelu(u, a_ref[0])
            sc = jnp.sum(g * w2_ref[...], axis=1, keepdims=True) + b2_ref[0]
            e = jnp.tanh(sc)
            if kind == 'e':
                orefs[o][...] = e
                o += 1
            else:
                m = hsv * e
                orefs[o][...] = m[:, 0:32]
                orefs[o + 1][...] = m[:, 32:64]
                orefs[o + 2][...] = m[:, 64:96]
                o += 3

    full = lambda shape: pl.BlockSpec(shape, lambda i: (0, 0))
    smem = pl.BlockSpec(memory_space=pltpu.MemorySpace.SMEM)
    in_specs = [pl.BlockSpec((BLK, 128), lambda i: (i, 0)),
                pl.BlockSpec((BLK, 128), lambda i: (i, 0))]
    args = [hs, hd]
    wspecs, sspecs, wargs, sargs = [], [], [], []
    out_specs, out_shape = [], []
    for kind, w1t, b1, a, w2row, b2 in heads:
        wspecs += [full((128, D)), full((1, D)), full((1, D))]
        wargs += [jnp.pad(w1t, ((0, 32), (0, 0))), b1, w2row]
        sspecs += [smem, smem]
        sargs += [a, b2]
        if kind == 'e':
            out_specs.append(pl.BlockSpec((BLK, 1), lambda i: (i, 0)))
            out_shape.append(jax.ShapeDtypeStruct((EPAD, 1), jnp.float32))
        else:
            for _ in range(3):
                out_specs.append(pl.BlockSpec((BLK, 32), lambda i: (i, 0)))
                out_shape.append(jax.ShapeDtypeStruct((EPAD, 32), jnp.float32))

    res = pl.pallas_call(
        body,
        grid=(EPAD // BLK,),
        in_specs=in_specs + wspecs + sspecs,
        out_specs=out_specs,
        out_shape=out_shape,
    )(*(args + wargs + sargs))
    return list(res) if isinstance(res, (list, tuple)) else [res]


def _node_call(zp, x, ln_g, ln_b, bn_scale, bn_b, act_a):
    BLK = 5000

    def body(zp_ref, x_ref, lng_ref, lnb_ref, bns_ref, bnb_ref, a_ref, out_ref):
        z = jnp.concatenate(
            [zp_ref[0, 0] + zp_ref[1, 0],
             zp_ref[0, 1] + zp_ref[1, 1],
             zp_ref[0, 2] + zp_ref[1, 2]], axis=1)
        y0 = z + x_ref[...]
        mu = jnp.mean(y0, axis=1, keepdims=True)
        d = y0 - mu
        var = jnp.mean(d * d, axis=1, keepdims=True)
        y = d * lax.rsqrt(var + 1e-5) * lng_ref[...] + lnb_ref[...]
        y = y * bns_ref[...] + bnb_ref[...]
        out_ref[...] = _prelu(y, a_ref[0])

    full = lambda shape: pl.BlockSpec(shape, lambda i: (0, 0))
    smem = pl.BlockSpec(memory_space=pltpu.MemorySpace.SMEM)
    return pl.pallas_call(
        body,
        grid=(N // BLK,),
        in_specs=[
            pl.BlockSpec((2, 3, BLK, 32), lambda i: (0, 0, i, 0)),
            pl.BlockSpec((BLK, D), lambda i: (i, 0)),
            full((1, D)), full((1, D)), full((1, D)), full((1, D)), smem,
        ],
        out_specs=pl.BlockSpec((BLK, D), lambda i: (i, 0)),
        out_shape=jax.ShapeDtypeStruct((N, D), jnp.float32),
    )(zp, x, ln_g, ln_b, bn_scale, bn_b, act_a)


def _head_call(raw, h1, h2, wa, wb, wc, bias):
    BLK = 5000
    OUT = 64

    def body(r_ref, h1_ref, h2_ref, wa_ref, wb_ref, wc_ref, b_ref, out_ref):
        acc = jnp.dot(r_ref[...], wa_ref[...], preferred_element_type=jnp.float32)
        acc += jnp.dot(h1_ref[...], wb_ref[...], preferred_element_type=jnp.float32)
        acc += jnp.dot(h2_ref[...], wc_ref[...], preferred_element_type=jnp.float32)
        out_ref[...] = acc + b_ref[...]

    full = lambda shape: pl.BlockSpec(shape, lambda i: (0, 0))
    return pl.pallas_call(
        body,
        grid=(N // BLK,),
        in_specs=[
            pl.BlockSpec((BLK, D), lambda i: (i, 0)),
            pl.BlockSpec((BLK, D), lambda i: (i, 0)),
            pl.BlockSpec((BLK, D), lambda i: (i, 0)),
            full((D, OUT)), full((D, OUT)), full((D, OUT)), full((1, OUT)),
        ],
        out_specs=pl.BlockSpec((BLK, OUT), lambda i: (i, 0)),
        out_shape=jax.ShapeDtypeStruct((N, OUT), jnp.float32),
    )(raw, h1, h2, wa, wb, wc, bias)


def _block_diag(blocks):
    n_in = sum(b.shape[0] for b in blocks)
    n_out = sum(b.shape[1] for b in blocks)
    out = jnp.zeros((n_in, n_out), jnp.float32)
    r = c = 0
    for b in blocks:
        out = lax.dynamic_update_slice(out, b, (r, c))
        r += b.shape[0]
        c += b.shape[1]
    return out


def kernel(h, edge_index, params):
    pad = jnp.full((EPAD - E,), DUMMY, jnp.int32)
    src1d = jnp.concatenate([edge_index[0], pad])
    dst1d = jnp.concatenate([edge_index[1], pad])

    hp = h[:, :, :, 0, :].reshape(25000, 192)
    hm = h[:, :, :, 1, :].reshape(25000, 192)

    def bd1(w):
        return _block_diag([w[g * 64:(g + 1) * 64, :].T for g in range(3)])

    def bd2(w):
        return _block_diag([w[g * 32:(g + 1) * 32, :].T for g in range(3)])

    row = lambda v: v.reshape(1, -1)
    sc = lambda v: v.reshape(1).astype(jnp.float32)

    xp, xm = _prologue_call(
        hp, hm,
        bd1(params['tp_w1']), row(params['tp_b1']), bd2(params['tp_w2']),
        row(params['tp_b2']),
        bd1(params['tm_w1']), row(params['tm_b1']), bd2(params['tm_w2']),
        row(params['tm_b2']),
        sc(params['tp_a']), sc(params['tm_a']))
    x0 = jnp.stack([xp, xm], axis=1).reshape(N, D)

    zeros_tile = jnp.zeros((ZPT, 32), jnp.float32)
    tpad = lambda x: jnp.pad(x, ((0, NPAD - N), (0, 128 - D)))

    def fa_head(lp, kind):
        fa = lp['fa']
        return (kind, fa['w1'].T, row(fa['b1']), sc(fa['a']),
                row(fa['w2'][0]), fa['b2'].astype(jnp.float32))

    def node_step(zp, x, lp):
        bn_scale = lp['bn_g'] / jnp.sqrt(1.0 + 1e-5)
        return _node_call(zp, x, row(lp['ln_g']), row(lp['ln_b']),
                          row(bn_scale), row(lp['bn_b']), sc(lp['act_a']))

    l0, l1 = params['layers'][0], params['layers'][1]

    hs0, hd0 = _sc_gather2(tpad(x0), src1d, dst1d)
    m0a, m0b, m0c = _edge_call(hs0, hd0, [fa_head(l0, 'm')])
    zp0 = _sc_scatter(m0a, m0b, m0c, dst1d, zeros_tile)
    x1 = node_step(zp0, x0, l0)

    hs1, hd1 = _sc_gather2(tpad(x1), src1d, dst1d)
    e0, m1a, m1b, m1c = _edge_call(hs1, hd1, [fa_head(l0, 'e'), fa_head(l1, 'm')])
    zp1 = _sc_scatter(m1a, m1b, m1c, dst1d, zeros_tile)
    x2 = node_step(zp1, x1, l1)

    hs2, hd2 = _sc_gather2(tpad(x2), src1d, dst1d)
    (e1,) = _edge_call(hs2, hd2, [fa_head(l1, 'e')])

    out = _head_call(x0, x1, x2,
                     params['t2_w'].T[0:D], params['t2_w'].T[D:2 * D],
                     params['t2_w'].T[2 * D:3 * D], row(params['t2_b']))
    ee = jnp.concatenate([e0[:E], e1[:E]], axis=0)
    return out, ee


# speedup vs baseline: 1.0968x; 1.0968x over previous
import functools

import jax
import jax.numpy as jnp
from jax import lax
from jax.experimental import pallas as pl
from jax.experimental.pallas import tpu as pltpu
from jax.experimental.pallas import tpu_sc as plsc

N = 50000
E = 800000
D = 96
NC = 2
NS = 16
CPT = 200
NCHUNK = 32 * CPT
EPAD = NCHUNK * 128
EPT = EPAD // 32
DUMMY = 50040
NPAD = 50048
ZPT = NPAD // NS
MG = 5
CHG = 800
NCHG = EPT // CHG

PERM = ([2 * j for j in range(32)] + [2 * j + 1 for j in range(32)]
        + [64 + 2 * j for j in range(16)] + [64 + 2 * j + 1 for j in range(16)])


@functools.cache
def _mesh():
    return plsc.VectorSubcoreMesh(core_axis_name="c", subcore_axis_name="s",
                                  num_cores=NC, num_subcores=NS)


def _sc_gather2(tblA, tblB, src1d, dst1d):
    return _make_gather2()(tblA, tblB, src1d, dst1d)


def _gather2_body(tblA, tblB, src1d, dst1d, hsA, hsB, hdA, hdB,
                  idx0, idx1, rA0, rB0, rA1, rB1,
                  gsem0, gsem1, wsem0, wsem1):
    c = lax.axis_index("c")
    s = lax.axis_index("s")
    wid = c * NS + s
    base0 = wid * EPT
    BUFS = ((idx0, rA0, rB0, gsem0, wsem0), (idx1, rA1, rB1, gsem1, wsem1))

    def phase(idx1d_hbm, outA, outB):
        def drain(bufA, bufB, sem):
            pltpu.make_async_copy(outA.at[pl.ds(0, CHG)], bufA, sem).wait()
            pltpu.make_async_copy(outB.at[pl.ds(0, CHG)], bufB, sem).wait()

        def stage(i, idxb, bufA, bufB, gsem):
            pltpu.sync_copy(idx1d_hbm.at[pl.ds(base0 + i * CHG, CHG)], idxb)
            pltpu.async_copy(tblA.at[idxb], bufA, gsem)
            pltpu.async_copy(tblB.at[idxb], bufB, gsem)

        stage(0, *BUFS[0][:4])

        def pair(j, carry):
            for b in range(2):
                i = j * 2 + b
                idxb, bufA, bufB, gsem, wsem = BUFS[b]
                nidxb, nbufA, nbufB, ngsem, nwsem = BUFS[1 - b]

                @pl.when(i + 1 < NCHG)
                def _():
                    @pl.when(i >= 1)
                    def _():
                        drain(nbufA, nbufB, nwsem)
                    stage(i + 1, nidxb, nbufA, nbufB, ngsem)

                drain(bufA, bufB, gsem)
                pltpu.async_copy(bufA, outA.at[pl.ds(base0 + i * CHG, CHG)],
                                 wsem)
                pltpu.async_copy(bufB, outB.at[pl.ds(base0 + i * CHG, CHG)],
                                 wsem)
            return carry

        lax.fori_loop(0, NCHG // 2, pair, 0)
        drain(rA0, rB0, wsem0)
        drain(rA1, rB1, wsem1)

    phase(src1d, hsA, hsB)
    phase(dst1d, hdA, hdB)


@functools.cache
def _make_gather2():
    return functools.partial(
        pl.kernel,
        out_type=(jax.ShapeDtypeStruct((EPAD, 32), jnp.uint32),
                  jax.ShapeDtypeStruct((EPAD, 16), jnp.uint32),
                  jax.ShapeDtypeStruct((EPAD, 32), jnp.uint32),
                  jax.ShapeDtypeStruct((EPAD, 16), jnp.uint32)),
        mesh=_mesh(),
        scratch_types=[
            pltpu.VMEM((CHG,), jnp.int32),
            pltpu.VMEM((CHG,), jnp.int32),
            pltpu.VMEM((CHG, 32), jnp.uint32),
            pltpu.VMEM((CHG, 16), jnp.uint32),
            pltpu.VMEM((CHG, 32), jnp.uint32),
            pltpu.VMEM((CHG, 16), jnp.uint32),
            pltpu.SemaphoreType.DMA,
            pltpu.SemaphoreType.DMA,
            pltpu.SemaphoreType.DMA,
            pltpu.SemaphoreType.DMA,
        ],
        compiler_params=pltpu.CompilerParams(use_tc_tiling_on_sc=False),
    )(_gather2_body)


def _sc_scatter(m0, m1, m2, dst1d, zeros_hbm):
    return _make_scatter()(m0, m1, m2, dst1d, zeros_hbm)


def _scatter_body(m0, m1, m2, dst1d, zeros_hbm, zp_out, didx, mbuf, zacc, sem):
    c = lax.axis_index("c")
    s = lax.axis_index("s")
    edge0 = (c * NS + s) * EPT

    for p, m_hbm in enumerate((m0, m1, m2)):
        pltpu.sync_copy(zeros_hbm, zacc.at[pl.ds(s * ZPT, ZPT)])
        plsc.subcore_barrier()

        def group(g, carry):
            b = edge0 + g * (MG * 128)
            pltpu.sync_copy(dst1d.at[pl.ds(b, MG * 128)], didx)
            pltpu.sync_copy(m_hbm.at[pl.ds(b, MG * 128)], mbuf)
            pltpu.sync_copy(mbuf, zacc.at[didx], add=True)
            return carry

        lax.fori_loop(0, EPT // (MG * 128), group, 0)
        plsc.subcore_barrier()

        @pl.when(s < NS - 1)
        def _():
            pltpu.sync_copy(zacc.at[pl.ds(s * ZPT, ZPT)],
                            zp_out.at[c, p, pl.ds(s * ZPT, ZPT)])

        @pl.when(s == NS - 1)
        def _():
            pltpu.sync_copy(
                zacc.at[pl.ds((NS - 1) * ZPT, N - (NS - 1) * ZPT)],
                zp_out.at[c, p, pl.ds((NS - 1) * ZPT, N - (NS - 1) * ZPT)])

        plsc.subcore_barrier()


@functools.cache
def _make_scatter():
    return functools.partial(
        pl.kernel,
        out_type=jax.ShapeDtypeStruct((2, 3, N, 32), jnp.float32),
        mesh=_mesh(),
        scratch_types=[
            pltpu.VMEM((MG * 128,), jnp.int32),
            pltpu.VMEM((MG * 128, 32), jnp.float32),
            pltpu.VMEM_SHARED((NPAD, 32), jnp.float32),
            pltpu.SemaphoreType.DMA,
        ],
        compiler_params=pltpu.CompilerParams(use_tc_tiling_on_sc=False),
    )(_scatter_body)


def _prelu(x, a):
    return jnp.where(x >= 0, x, a * x)


def _pack_pair(lo, hi):
    ul = lax.bitcast_convert_type(lo, jnp.uint32)
    uh = lax.bitcast_convert_type(hi, jnp.uint32)
    rl = (ul + 0x7FFF + ((ul >> 16) & 1)) >> 16
    rh = (uh + 0x7FFF + ((uh >> 16) & 1)) & jnp.uint32(0xFFFF0000)
    return rh | rl


def _unpack_lo(w):
    return lax.bitcast_convert_type(w << 16, jnp.float32)


def _unpack_hi(w):
    return lax.bitcast_convert_type(w & jnp.uint32(0xFFFF0000), jnp.float32)


def _pack_cols(xP):
    pkA = _pack_pair(xP[:, 0:32], xP[:, 32:64])
    pkB = _pack_pair(xP[:, 64:80], xP[:, 80:96])
    return pkA, pkB


def _prologue_call(hp, hm, wp1, bp1, wp2, bp2, wm1, bm1, wm2, bm2, ap, am):
    R, BLK = 25000, 5000

    def body(hp_ref, hm_ref, wp1_ref, bp1_ref, wp2_ref, bp2_ref,
             wm1_ref, bm1_ref, wm2_ref, bm2_ref, ap_ref, am_ref,
             xp_ref, xm_ref):
        t = jnp.dot(hp_ref[...], wp1_ref[...],
                    preferred_element_type=jnp.float32) + bp1_ref[...]
        t = _prelu(t, ap_ref[0])
        xp_ref[...] = jnp.dot(t, wp2_ref[...],
                              preferred_element_type=jnp.float32) + bp2_ref[...]
        u = jnp.dot(hm_ref[...], wm1_ref[...],
                    preferred_element_type=jnp.float32) + bm1_ref[...]
        u = _prelu(u, am_ref[0])
        xm_ref[...] = jnp.dot(u, wm2_ref[...],
                              preferred_element_type=jnp.float32) + bm2_ref[...]

    full = lambda shape: pl.BlockSpec(shape, lambda i: (0, 0))
    smem = pl.BlockSpec(memory_space=pltpu.MemorySpace.SMEM)
    return pl.pallas_call(
        body,
        grid=(R // BLK,),
        in_specs=[
            pl.BlockSpec((BLK, 192), lambda i: (i, 0)),
            pl.BlockSpec((BLK, 192), lambda i: (i, 0)),
            full((192, 192)), full((1, 192)), full((192, D)), full((1, D)),
            full((192, 192)), full((1, 192)), full((192, D)), full((1, D)),
            smem, smem,
        ],
        out_specs=[pl.BlockSpec((BLK, D), lambda i: (i, 0)),
                   pl.BlockSpec((BLK, D), lambda i: (i, 0))],
        out_shape=[jax.ShapeDtypeStruct((R, D), jnp.float32),
                   jax.ShapeDtypeStruct((R, D), jnp.float32)],
    )(hp, hm, wp1, bp1, wp2, bp2, wm1, bm1, wm2, bm2, ap, am)


def _pack_call(x):
    BLK = 5000

    def body(x_ref, a_ref, b_ref):
        pkA, pkB = _pack_cols(x_ref[...])
        a_ref[...] = pkA
        b_ref[...] = pkB

    return pl.pallas_call(
        body,
        grid=(N // BLK,),
        in_specs=[pl.BlockSpec((BLK, D), lambda i: (i, 0))],
        out_specs=[pl.BlockSpec((BLK, 32), lambda i: (i, 0)),
                   pl.BlockSpec((BLK, 16), lambda i: (i, 0))],
        out_shape=[jax.ShapeDtypeStruct((N, 32), jnp.uint32),
                   jax.ShapeDtypeStruct((N, 16), jnp.uint32)],
    )(x)


def _edge_call(hsA, hsB, hdA, hdB, heads):
    BLK = 4096

    def body(*refs):
        hsA_ref, hsB_ref, hdA_ref, hdB_ref = refs[0:4]
        wrefs = refs[4:4 + 3 * len(heads)]
        srefs = refs[4 + 3 * len(heads):4 + 5 * len(heads)]
        orefs = refs[4 + 5 * len(heads):]
        wsA = hsA_ref[...]
        wsB = hsB_ref[...]
        wdA = hdA_ref[...]
        wdB = hdB_ref[...]
        sparts = (_unpack_lo(wsA), _unpack_hi(wsA),
                  _unpack_lo(wsB), _unpack_hi(wsB))
        dparts = (_unpack_lo(wdA), _unpack_hi(wdA),
                  _unpack_lo(wdB), _unpack_hi(wdB))
        h2 = jnp.concatenate([a * b for a, b in zip(sparts, dparts)], axis=1)
        hsv = None
        if any(head[0] == 'm' for head in heads):
            hsv = jnp.concatenate(sparts, axis=1)
        o = 0
        for i, head in enumerate(heads):
            kind = head[0]
            w1t_ref, b1_ref, w2_ref = wrefs[3 * i], wrefs[3 * i + 1], wrefs[3 * i + 2]
            a_ref, b2_ref = srefs[2 * i], srefs[2 * i + 1]
            u = jnp.dot(h2, w1t_ref[...],
                        preferred_element_type=jnp.float32) + b1_ref[...]
            g = _prelu(u, a_ref[0])
            sc = jnp.sum(g * w2_ref[...], axis=1, keepdims=True) + b2_ref[0]
            e = jnp.tanh(sc)
            if kind == 'e':
                orefs[o][...] = e
                o += 1
            else:
                m = hsv * e
                orefs[o][...] = m[:, 0:32]
                orefs[o + 1][...] = m[:, 32:64]
                orefs[o + 2][...] = m[:, 64:96]
                o += 3

    full = lambda shape: pl.BlockSpec(shape, lambda i: (0, 0))
    smem = pl.BlockSpec(memory_space=pltpu.MemorySpace.SMEM)
    in_specs = [pl.BlockSpec((BLK, 32), lambda i: (i, 0)),
                pl.BlockSpec((BLK, 16), lambda i: (i, 0)),
                pl.BlockSpec((BLK, 32), lambda i: (i, 0)),
                pl.BlockSpec((BLK, 16), lambda i: (i, 0))]
    args = [hsA, hsB, hdA, hdB]
    wspecs, sspecs, wargs, sargs = [], [], [], []
    out_specs, out_shape = [], []
    for kind, w1t, b1, a, w2row, b2 in heads:
        wspecs += [full((D, D)), full((1, D)), full((1, D))]
        wargs += [w1t, b1, w2row]
        sspecs += [smem, smem]
        sargs += [a, b2]
        if kind == 'e':
            out_specs.append(pl.BlockSpec((BLK, 1), lambda i: (i, 0)))
            out_shape.append(jax.ShapeDtypeStruct((EPAD, 1), jnp.float32))
        else:
            for _ in range(3):
                out_specs.append(pl.BlockSpec((BLK, 32), lambda i: (i, 0)))
                out_shape.append(jax.ShapeDtypeStruct((EPAD, 32), jnp.float32))

    res = pl.pallas_call(
        body,
        grid=(EPAD // BLK,),
        in_specs=in_specs + wspecs + sspecs,
        out_specs=out_specs,
        out_shape=out_shape,
    )(*(args + wargs + sargs))
    return list(res) if isinstance(res, (list, tuple)) else [res]


def _node_call(zp, x, ln_g, ln_b, bn_scale, bn_b, act_a):
    BLK = 5000

    def body(zp_ref, x_ref, lng_ref, lnb_ref, bns_ref, bnb_ref, a_ref,
             out_ref, pa_ref, pb_ref):
        z = jnp.concatenate(
            [zp_ref[0, 0] + zp_ref[1, 0],
             zp_ref[0, 1] + zp_ref[1, 1],
             zp_ref[0, 2] + zp_ref[1, 2]], axis=1)
        y0 = z + x_ref[...]
        mu = jnp.mean(y0, axis=1, keepdims=True)
        d = y0 - mu
        var = jnp.mean(d * d, axis=1, keepdims=True)
        y = d * lax.rsqrt(var + 1e-5) * lng_ref[...] + lnb_ref[...]
        y = y * bns_ref[...] + bnb_ref[...]
        xn = _prelu(y, a_ref[0])
        out_ref[...] = xn
        pkA, pkB = _pack_cols(xn)
        pa_ref[...] = pkA
        pb_ref[...] = pkB

    full = lambda shape: pl.BlockSpec(shape, lambda i: (0, 0))
    smem = pl.BlockSpec(memory_space=pltpu.MemorySpace.SMEM)
    return pl.pallas_call(
        body,
        grid=(N // BLK,),
        in_specs=[
            pl.BlockSpec((2, 3, BLK, 32), lambda i: (0, 0, i, 0)),
            pl.BlockSpec((BLK, D), lambda i: (i, 0)),
            full((1, D)), full((1, D)), full((1, D)), full((1, D)), smem,
        ],
        out_specs=[pl.BlockSpec((BLK, D), lambda i: (i, 0)),
                   pl.BlockSpec((BLK, 32), lambda i: (i, 0)),
                   pl.BlockSpec((BLK, 16), lambda i: (i, 0))],
        out_shape=[jax.ShapeDtypeStruct((N, D), jnp.float32),
                   jax.ShapeDtypeStruct((N, 32), jnp.uint32),
                   jax.ShapeDtypeStruct((N, 16), jnp.uint32)],
    )(zp, x, ln_g, ln_b, bn_scale, bn_b, act_a)


def _head_call(raw, h1, h2, wa, wb, wc, bias):
    BLK = 5000
    OUT = 64

    def body(r_ref, h1_ref, h2_ref, wa_ref, wb_ref, wc_ref, b_ref, out_ref):
        acc = jnp.dot(r_ref[...], wa_ref[...], preferred_element_type=jnp.float32)
        acc += jnp.dot(h1_ref[...], wb_ref[...], preferred_element_type=jnp.float32)
        acc += jnp.dot(h2_ref[...], wc_ref[...], preferred_element_type=jnp.float32)
        out_ref[...] = acc + b_ref[...]

    full = lambda shape: pl.BlockSpec(shape, lambda i: (0, 0))
    return pl.pallas_call(
        body,
        grid=(N // BLK,),
        in_specs=[
            pl.BlockSpec((BLK, D), lambda i: (i, 0)),
            pl.BlockSpec((BLK, D), lambda i: (i, 0)),
            pl.BlockSpec((BLK, D), lambda i: (i, 0)),
            full((D, OUT)), full((D, OUT)), full((D, OUT)), full((1, OUT)),
        ],
        out_specs=pl.BlockSpec((BLK, OUT), lambda i: (i, 0)),
        out_shape=jax.ShapeDtypeStruct((N, OUT), jnp.float32),
    )(raw, h1, h2, wa, wb, wc, bias)


def _block_diag(blocks):
    n_in = sum(b.shape[0] for b in blocks)
    n_out = sum(b.shape[1] for b in blocks)
    out = jnp.zeros((n_in, n_out), jnp.float32)
    r = c = 0
    for b in blocks:
        out = lax.dynamic_update_slice(out, b, (r, c))
        r += b.shape[0]
        c += b.shape[1]
    return out


def kernel(h, edge_index, params):
    perm = jnp.asarray(PERM, jnp.int32)
    pad = jnp.full((EPAD - E,), DUMMY, jnp.int32)
    src1d = jnp.concatenate([edge_index[0], pad])
    dst1d = jnp.concatenate([edge_index[1], pad])

    hp = h[:, :, :, 0, :].reshape(25000, 192)
    hm = h[:, :, :, 1, :].reshape(25000, 192)

    def bd1(w):
        return _block_diag([w[g * 64:(g + 1) * 64, :].T for g in range(3)])

    def bd2(w):
        return _block_diag([w[g * 32:(g + 1) * 32, :].T
                            for g in range(3)])[:, perm]

    row = lambda v: v.reshape(1, -1)
    sc = lambda v: v.reshape(1).astype(jnp.float32)

    xp, xm = _prologue_call(
        hp, hm,
        bd1(params['tp_w1']), row(params['tp_b1']), bd2(params['tp_w2']),
        row(params['tp_b2'][perm]),
        bd1(params['tm_w1']), row(params['tm_b1']), bd2(params['tm_w2']),
        row(params['tm_b2'][perm]),
        sc(params['tp_a']), sc(params['tm_a']))
    x0 = jnp.stack([xp, xm], axis=1).reshape(N, D)
    x0pkA, x0pkB = _pack_call(x0)

    zeros_tile = jnp.zeros((ZPT, 32), jnp.float32)
    tpadA = lambda t: jnp.pad(t, ((0, NPAD - N), (0, 0)))

    def fa_head(lp, kind):
        fa = lp['fa']
        return (kind, fa['w1'].T[perm, :], row(fa['b1']), sc(fa['a']),
                row(fa['w2'][0]), fa['b2'].astype(jnp.float32))

    def node_step(zp, x, lp):
        bn_scale = (lp['bn_g'] / jnp.sqrt(1.0 + 1e-5))[perm]
        return _node_call(zp, x, row(lp['ln_g'][perm]), row(lp['ln_b'][perm]),
                          row(bn_scale), row(lp['bn_b'][perm]),
                          sc(lp['act_a']))

    l0, l1 = params['layers'][0], params['layers'][1]

    hsA0, hsB0, hdA0, hdB0 = _sc_gather2(tpadA(x0pkA), tpadA(x0pkB),
                                         src1d, dst1d)
    m0a, m0b, m0c = _edge_call(hsA0, hsB0, hdA0, hdB0, [fa_head(l0, 'm')])
    zp0 = _sc_scatter(m0a, m0b, m0c, dst1d, zeros_tile)
    x1, x1pkA, x1pkB = node_step(zp0, x0, l0)

    hsA1, hsB1, hdA1, hdB1 = _sc_gather2(tpadA(x1pkA), tpadA(x1pkB),
                                         src1d, dst1d)
    e0, m1a, m1b, m1c = _edge_call(hsA1, hsB1, hdA1, hdB1,
                                   [fa_head(l0, 'e'), fa_head(l1, 'm')])
    zp1 = _sc_scatter(m1a, m1b, m1c, dst1d, zeros_tile)
    x2, x2pkA, x2pkB = node_step(zp1, x1, l1)

    hsA2, hsB2, hdA2, hdB2 = _sc_gather2(tpadA(x2pkA), tpadA(x2pkB),
                                         src1d, dst1d)
    (e1,) = _edge_call(hsA2, hsB2, hdA2, hdB2, [fa_head(l1, 'e')])

    w2t = params['t2_w'].T
    out = _head_call(x0, x1, x2,
                     w2t[0:D][perm, :], w2t[D:2 * D][perm, :],
                     w2t[2 * D:3 * D][perm, :], row(params['t2_b']))
    ee = jnp.concatenate([e0[:E], e1[:E]], axis=0)
    return out, ee


# speedup vs baseline: 1.1122x; 1.0140x over previous
import functools

import jax
import jax.numpy as jnp
from jax import lax
from jax.experimental import pallas as pl
from jax.experimental.pallas import tpu as pltpu
from jax.experimental.pallas import tpu_sc as plsc

N = 50000
E = 800000
D = 96
NC = 2
NS = 16
CPT = 200
NCHUNK = 32 * CPT
EPAD = NCHUNK * 128
EPT = EPAD // 32
DUMMY = 50040
NPAD = 50048
ZPT = NPAD // NS
MG = 5
CHG = 800
NCHG = EPT // CHG

PERM = ([2 * j for j in range(32)] + [2 * j + 1 for j in range(32)]
        + [64 + 2 * j for j in range(16)] + [64 + 2 * j + 1 for j in range(16)])


@functools.cache
def _mesh():
    return plsc.VectorSubcoreMesh(core_axis_name="c", subcore_axis_name="s",
                                  num_cores=NC, num_subcores=NS)


def _sc_gather2(tblA, tblB, src1d, dst1d):
    return _make_gather2()(tblA, tblB, src1d, dst1d)


def _gather2_body(tblA, tblB, src1d, dst1d, hsA, hsB, hdA, hdB,
                  idx0, idx1, rA0, rB0, rA1, rB1,
                  gsem0, gsem1, wsem0, wsem1):
    c = lax.axis_index("c")
    s = lax.axis_index("s")
    wid = c * NS + s
    base0 = wid * EPT
    BUFS = ((idx0, rA0, rB0, gsem0, wsem0), (idx1, rA1, rB1, gsem1, wsem1))

    def phase(idx1d_hbm, outA, outB):
        def drain(bufA, bufB, sem):
            pltpu.make_async_copy(outA.at[pl.ds(0, CHG)], bufA, sem).wait()
            pltpu.make_async_copy(outB.at[pl.ds(0, CHG)], bufB, sem).wait()

        def stage(i, idxb, bufA, bufB, gsem):
            pltpu.sync_copy(idx1d_hbm.at[pl.ds(base0 + i * CHG, CHG)], idxb)
            pltpu.async_copy(tblA.at[idxb], bufA, gsem)
            pltpu.async_copy(tblB.at[idxb], bufB, gsem)

        stage(0, *BUFS[0][:4])

        def pair(j, carry):
            for b in range(2):
                i = j * 2 + b
                idxb, bufA, bufB, gsem, wsem = BUFS[b]
                nidxb, nbufA, nbufB, ngsem, nwsem = BUFS[1 - b]

                @pl.when(i + 1 < NCHG)
                def _():
                    @pl.when(i >= 1)
                    def _():
                        drain(nbufA, nbufB, nwsem)
                    stage(i + 1, nidxb, nbufA, nbufB, ngsem)

                drain(bufA, bufB, gsem)
                pltpu.async_copy(bufA, outA.at[pl.ds(base0 + i * CHG, CHG)],
                                 wsem)
                pltpu.async_copy(bufB, outB.at[pl.ds(base0 + i * CHG, CHG)],
                                 wsem)
            return carry

        lax.fori_loop(0, NCHG // 2, pair, 0)
        drain(rA0, rB0, wsem0)
        drain(rA1, rB1, wsem1)

    phase(src1d, hsA, hsB)
    phase(dst1d, hdA, hdB)


@functools.cache
def _make_gather2():
    return functools.partial(
        pl.kernel,
        out_type=(jax.ShapeDtypeStruct((EPAD, 32), jnp.uint32),
                  jax.ShapeDtypeStruct((EPAD, 16), jnp.uint32),
                  jax.ShapeDtypeStruct((EPAD, 32), jnp.uint32),
                  jax.ShapeDtypeStruct((EPAD, 16), jnp.uint32)),
        mesh=_mesh(),
        scratch_types=[
            pltpu.VMEM((CHG,), jnp.int32),
            pltpu.VMEM((CHG,), jnp.int32),
            pltpu.VMEM((CHG, 32), jnp.uint32),
            pltpu.VMEM((CHG, 16), jnp.uint32),
            pltpu.VMEM((CHG, 32), jnp.uint32),
            pltpu.VMEM((CHG, 16), jnp.uint32),
            pltpu.SemaphoreType.DMA,
            pltpu.SemaphoreType.DMA,
            pltpu.SemaphoreType.DMA,
            pltpu.SemaphoreType.DMA,
        ],
        compiler_params=pltpu.CompilerParams(use_tc_tiling_on_sc=False),
    )(_gather2_body)


def _sc_scatter(m0, m1, m2, dst1d, zeros_hbm):
    return _make_scatter()(m0, m1, m2, dst1d, zeros_hbm)


def _scatter_body(m0, m1, m2, dst1d, zeros_hbm, zp_out, didx, mbuf, zacc, sem):
    c = lax.axis_index("c")
    s = lax.axis_index("s")
    edge0 = (c * NS + s) * EPT

    for p, m_hbm in enumerate((m0, m1, m2)):
        pltpu.sync_copy(zeros_hbm, zacc.at[pl.ds(s * ZPT, ZPT)])
        plsc.subcore_barrier()

        def group(g, carry):
            b = edge0 + g * (MG * 128)
            pltpu.sync_copy(dst1d.at[pl.ds(b, MG * 128)], didx)
            pltpu.sync_copy(m_hbm.at[pl.ds(b, MG * 128)], mbuf)
            pltpu.sync_copy(mbuf, zacc.at[didx], add=True)
            return carry

        lax.fori_loop(0, EPT // (MG * 128), group, 0)
        plsc.subcore_barrier()

        @pl.when(s < NS - 1)
        def _():
            pltpu.sync_copy(zacc.at[pl.ds(s * ZPT, ZPT)],
                            zp_out.at[c, p, pl.ds(s * ZPT, ZPT)])

        @pl.when(s == NS - 1)
        def _():
            pltpu.sync_copy(
                zacc.at[pl.ds((NS - 1) * ZPT, N - (NS - 1) * ZPT)],
                zp_out.at[c, p, pl.ds((NS - 1) * ZPT, N - (NS - 1) * ZPT)])

        plsc.subcore_barrier()


@functools.cache
def _make_scatter():
    return functools.partial(
        pl.kernel,
        out_type=jax.ShapeDtypeStruct((2, 3, N, 32), jnp.float32),
        mesh=_mesh(),
        scratch_types=[
            pltpu.VMEM((MG * 128,), jnp.int32),
            pltpu.VMEM((MG * 128, 32), jnp.float32),
            pltpu.VMEM_SHARED((NPAD, 32), jnp.float32),
            pltpu.SemaphoreType.DMA,
        ],
        compiler_params=pltpu.CompilerParams(use_tc_tiling_on_sc=False),
    )(_scatter_body)


def _prelu(x, a):
    return jnp.where(x >= 0, x, a * x)


def _pack_pair(lo, hi):
    ul = lax.bitcast_convert_type(lo, jnp.uint32)
    uh = lax.bitcast_convert_type(hi, jnp.uint32)
    rl = (ul + 0x7FFF + ((ul >> 16) & 1)) >> 16
    rh = (uh + 0x7FFF + ((uh >> 16) & 1)) & jnp.uint32(0xFFFF0000)
    return rh | rl


def _unpack_lo(w):
    return lax.bitcast_convert_type(w << 16, jnp.float32)


def _unpack_hi(w):
    return lax.bitcast_convert_type(w & jnp.uint32(0xFFFF0000), jnp.float32)


def _pack_cols(xP):
    pkA = _pack_pair(xP[:, 0:32], xP[:, 32:64])
    pkB = _pack_pair(xP[:, 64:80], xP[:, 80:96])
    return pkA, pkB


def _prologue_call(hp, hm, wp1, bp1, wp2, bp2, wm1, bm1, wm2, bm2, ap, am):
    R, BLK = 25000, 5000

    def body(hp_ref, hm_ref, wp1_ref, bp1_ref, wp2_ref, bp2_ref,
             wm1_ref, bm1_ref, wm2_ref, bm2_ref, ap_ref, am_ref,
             xp_ref, xm_ref):
        t = jnp.dot(hp_ref[...], wp1_ref[...],
                    preferred_element_type=jnp.float32) + bp1_ref[...]
        t = _prelu(t, ap_ref[0])
        xp_ref[...] = jnp.dot(t, wp2_ref[...],
                              preferred_element_type=jnp.float32) + bp2_ref[...]
        u = jnp.dot(hm_ref[...], wm1_ref[...],
                    preferred_element_type=jnp.float32) + bm1_ref[...]
        u = _prelu(u, am_ref[0])
        xm_ref[...] = jnp.dot(u, wm2_ref[...],
                              preferred_element_type=jnp.float32) + bm2_ref[...]

    full = lambda shape: pl.BlockSpec(shape, lambda i: (0, 0))
    smem = pl.BlockSpec(memory_space=pltpu.MemorySpace.SMEM)
    return pl.pallas_call(
        body,
        grid=(R // BLK,),
        in_specs=[
            pl.BlockSpec((BLK, 192), lambda i: (i, 0)),
            pl.BlockSpec((BLK, 192), lambda i: (i, 0)),
            full((192, 192)), full((1, 192)), full((192, D)), full((1, D)),
            full((192, 192)), full((1, 192)), full((192, D)), full((1, D)),
            smem, smem,
        ],
        out_specs=[pl.BlockSpec((BLK, D), lambda i: (i, 0)),
                   pl.BlockSpec((BLK, D), lambda i: (i, 0))],
        out_shape=[jax.ShapeDtypeStruct((R, D), jnp.float32),
                   jax.ShapeDtypeStruct((R, D), jnp.float32)],
    )(hp, hm, wp1, bp1, wp2, bp2, wm1, bm1, wm2, bm2, ap, am)


def _pack_call(x):
    BLK = 5000

    def body(x_ref, a_ref, b_ref):
        pkA, pkB = _pack_cols(x_ref[...])
        a_ref[...] = pkA
        b_ref[...] = pkB

    return pl.pallas_call(
        body,
        grid=(N // BLK,),
        in_specs=[pl.BlockSpec((BLK, D), lambda i: (i, 0))],
        out_specs=[pl.BlockSpec((BLK, 32), lambda i: (i, 0)),
                   pl.BlockSpec((BLK, 16), lambda i: (i, 0))],
        out_shape=[jax.ShapeDtypeStruct((N, 32), jnp.uint32),
                   jax.ShapeDtypeStruct((N, 16), jnp.uint32)],
    )(x)


def _edge_call(hsA, hsB, hdA, hdB, heads):
    BLK = 4096

    def body(*refs):
        hsA_ref, hsB_ref, hdA_ref, hdB_ref = refs[0:4]
        wrefs = refs[4:4 + 3 * len(heads)]
        srefs = refs[4 + 3 * len(heads):4 + 5 * len(heads)]
        orefs = refs[4 + 5 * len(heads):]
        wsA = hsA_ref[...]
        wsB = hsB_ref[...]
        wdA = hdA_ref[...]
        wdB = hdB_ref[...]
        sparts = (_unpack_lo(wsA), _unpack_hi(wsA),
                  _unpack_lo(wsB), _unpack_hi(wsB))
        dparts = (_unpack_lo(wdA), _unpack_hi(wdA),
                  _unpack_lo(wdB), _unpack_hi(wdB))
        h2_scr = refs[-1]
        h2_scr[:, 0:32] = sparts[0] * dparts[0]
        h2_scr[:, 32:64] = sparts[1] * dparts[1]
        h2_scr[:, 64:80] = sparts[2] * dparts[2]
        h2_scr[:, 80:96] = sparts[3] * dparts[3]
        h2 = h2_scr[...]
        o = 0
        for i, head in enumerate(heads):
            kind = head[0]
            w1t_ref, b1_ref, w2_ref = wrefs[3 * i], wrefs[3 * i + 1], wrefs[3 * i + 2]
            a_ref, b2_ref = srefs[2 * i], srefs[2 * i + 1]
            u = jnp.dot(h2, w1t_ref[...],
                        preferred_element_type=jnp.float32) + b1_ref[...]
            g = _prelu(u, a_ref[0])
            sc = jnp.sum(g * w2_ref[...], axis=1, keepdims=True) + b2_ref[0]
            e = jnp.tanh(sc)
            if kind == 'e':
                orefs[o][...] = e
                o += 1
            else:
                orefs[o][...] = sparts[0] * e
                orefs[o + 1][...] = sparts[1] * e
                orefs[o + 2][:, 0:16] = sparts[2] * e
                orefs[o + 2][:, 16:32] = sparts[3] * e
                o += 3

    full = lambda shape: pl.BlockSpec(shape, lambda i: (0, 0))
    smem = pl.BlockSpec(memory_space=pltpu.MemorySpace.SMEM)
    in_specs = [pl.BlockSpec((BLK, 32), lambda i: (i, 0)),
                pl.BlockSpec((BLK, 16), lambda i: (i, 0)),
                pl.BlockSpec((BLK, 32), lambda i: (i, 0)),
                pl.BlockSpec((BLK, 16), lambda i: (i, 0))]
    args = [hsA, hsB, hdA, hdB]
    wspecs, sspecs, wargs, sargs = [], [], [], []
    out_specs, out_shape = [], []
    for kind, w1t, b1, a, w2row, b2 in heads:
        wspecs += [full((D, D)), full((1, D)), full((1, D))]
        wargs += [w1t, b1, w2row]
        sspecs += [smem, smem]
        sargs += [a, b2]
        if kind == 'e':
            out_specs.append(pl.BlockSpec((BLK, 1), lambda i: (i, 0)))
            out_shape.append(jax.ShapeDtypeStruct((EPAD, 1), jnp.float32))
        else:
            for _ in range(3):
                out_specs.append(pl.BlockSpec((BLK, 32), lambda i: (i, 0)))
                out_shape.append(jax.ShapeDtypeStruct((EPAD, 32), jnp.float32))

    res = pl.pallas_call(
        body,
        grid=(EPAD // BLK,),
        in_specs=in_specs + wspecs + sspecs,
        out_specs=out_specs,
        out_shape=out_shape,
        scratch_shapes=[pltpu.VMEM((BLK, D), jnp.float32)],
    )(*(args + wargs + sargs))
    return list(res) if isinstance(res, (list, tuple)) else [res]


def _node_call(zp, x, ln_g, ln_b, bn_scale, bn_b, act_a):
    BLK = 5000

    def body(zp_ref, x_ref, lng_ref, lnb_ref, bns_ref, bnb_ref, a_ref,
             out_ref, pa_ref, pb_ref):
        z = jnp.concatenate(
            [zp_ref[0, 0] + zp_ref[1, 0],
             zp_ref[0, 1] + zp_ref[1, 1],
             zp_ref[0, 2] + zp_ref[1, 2]], axis=1)
        y0 = z + x_ref[...]
        mu = jnp.mean(y0, axis=1, keepdims=True)
        d = y0 - mu
        var = jnp.mean(d * d, axis=1, keepdims=True)
        y = d * lax.rsqrt(var + 1e-5) * lng_ref[...] + lnb_ref[...]
        y = y * bns_ref[...] + bnb_ref[...]
        xn = _prelu(y, a_ref[0])
        out_ref[...] = xn
        pkA, pkB = _pack_cols(xn)
        pa_ref[...] = pkA
        pb_ref[...] = pkB

    full = lambda shape: pl.BlockSpec(shape, lambda i: (0, 0))
    smem = pl.BlockSpec(memory_space=pltpu.MemorySpace.SMEM)
    return pl.pallas_call(
        body,
        grid=(N // BLK,),
        in_specs=[
            pl.BlockSpec((2, 3, BLK, 32), lambda i: (0, 0, i, 0)),
            pl.BlockSpec((BLK, D), lambda i: (i, 0)),
            full((1, D)), full((1, D)), full((1, D)), full((1, D)), smem,
        ],
        out_specs=[pl.BlockSpec((BLK, D), lambda i: (i, 0)),
                   pl.BlockSpec((BLK, 32), lambda i: (i, 0)),
                   pl.BlockSpec((BLK, 16), lambda i: (i, 0))],
        out_shape=[jax.ShapeDtypeStruct((N, D), jnp.float32),
                   jax.ShapeDtypeStruct((N, 32), jnp.uint32),
                   jax.ShapeDtypeStruct((N, 16), jnp.uint32)],
    )(zp, x, ln_g, ln_b, bn_scale, bn_b, act_a)


def _head_call(raw, h1, h2, wa, wb, wc, bias):
    BLK = 5000
    OUT = 64

    def body(r_ref, h1_ref, h2_ref, wa_ref, wb_ref, wc_ref, b_ref, out_ref):
        acc = jnp.dot(r_ref[...], wa_ref[...], preferred_element_type=jnp.float32)
        acc += jnp.dot(h1_ref[...], wb_ref[...], preferred_element_type=jnp.float32)
        acc += jnp.dot(h2_ref[...], wc_ref[...], preferred_element_type=jnp.float32)
        out_ref[...] = acc + b_ref[...]

    full = lambda shape: pl.BlockSpec(shape, lambda i: (0, 0))
    return pl.pallas_call(
        body,
        grid=(N // BLK,),
        in_specs=[
            pl.BlockSpec((BLK, D), lambda i: (i, 0)),
            pl.BlockSpec((BLK, D), lambda i: (i, 0)),
            pl.BlockSpec((BLK, D), lambda i: (i, 0)),
            full((D, OUT)), full((D, OUT)), full((D, OUT)), full((1, OUT)),
        ],
        out_specs=pl.BlockSpec((BLK, OUT), lambda i: (i, 0)),
        out_shape=jax.ShapeDtypeStruct((N, OUT), jnp.float32),
    )(raw, h1, h2, wa, wb, wc, bias)


def _block_diag(blocks):
    n_in = sum(b.shape[0] for b in blocks)
    n_out = sum(b.shape[1] for b in blocks)
    out = jnp.zeros((n_in, n_out), jnp.float32)
    r = c = 0
    for b in blocks:
        out = lax.dynamic_update_slice(out, b, (r, c))
        r += b.shape[0]
        c += b.shape[1]
    return out


def kernel(h, edge_index, params):
    perm = jnp.asarray(PERM, jnp.int32)
    pad = jnp.full((EPAD - E,), DUMMY, jnp.int32)
    src1d = jnp.concatenate([edge_index[0], pad])
    dst1d = jnp.concatenate([edge_index[1], pad])

    hp = h[:, :, :, 0, :].reshape(25000, 192)
    hm = h[:, :, :, 1, :].reshape(25000, 192)

    def bd1(w):
        return _block_diag([w[g * 64:(g + 1) * 64, :].T for g in range(3)])

    def bd2(w):
        return _block_diag([w[g * 32:(g + 1) * 32, :].T
                            for g in range(3)])[:, perm]

    row = lambda v: v.reshape(1, -1)
    sc = lambda v: v.reshape(1).astype(jnp.float32)

    xp, xm = _prologue_call(
        hp, hm,
        bd1(params['tp_w1']), row(params['tp_b1']), bd2(params['tp_w2']),
        row(params['tp_b2'][perm]),
        bd1(params['tm_w1']), row(params['tm_b1']), bd2(params['tm_w2']),
        row(params['tm_b2'][perm]),
        sc(params['tp_a']), sc(params['tm_a']))
    x0 = jnp.stack([xp, xm], axis=1).reshape(N, D)
    x0pkA, x0pkB = _pack_call(x0)

    zeros_tile = jnp.zeros((ZPT, 32), jnp.float32)
    tpadA = lambda t: jnp.pad(t, ((0, NPAD - N), (0, 0)))

    def fa_head(lp, kind):
        fa = lp['fa']
        return (kind, fa['w1'].T[perm, :], row(fa['b1']), sc(fa['a']),
                row(fa['w2'][0]), fa['b2'].astype(jnp.float32))

    def node_step(zp, x, lp):
        bn_scale = (lp['bn_g'] / jnp.sqrt(1.0 + 1e-5))[perm]
        return _node_call(zp, x, row(lp['ln_g'][perm]), row(lp['ln_b'][perm]),
                          row(bn_scale), row(lp['bn_b'][perm]),
                          sc(lp['act_a']))

    l0, l1 = params['layers'][0], params['layers'][1]

    hsA0, hsB0, hdA0, hdB0 = _sc_gather2(tpadA(x0pkA), tpadA(x0pkB),
                                         src1d, dst1d)
    m0a, m0b, m0c = _edge_call(hsA0, hsB0, hdA0, hdB0, [fa_head(l0, 'm')])
    zp0 = _sc_scatter(m0a, m0b, m0c, dst1d, zeros_tile)
    x1, x1pkA, x1pkB = node_step(zp0, x0, l0)

    hsA1, hsB1, hdA1, hdB1 = _sc_gather2(tpadA(x1pkA), tpadA(x1pkB),
                                         src1d, dst1d)
    e0, m1a, m1b, m1c = _edge_call(hsA1, hsB1, hdA1, hdB1,
                                   [fa_head(l0, 'e'), fa_head(l1, 'm')])
    zp1 = _sc_scatter(m1a, m1b, m1c, dst1d, zeros_tile)
    x2, x2pkA, x2pkB = node_step(zp1, x1, l1)

    hsA2, hsB2, hdA2, hdB2 = _sc_gather2(tpadA(x2pkA), tpadA(x2pkB),
                                         src1d, dst1d)
    (e1,) = _edge_call(hsA2, hsB2, hdA2, hdB2, [fa_head(l1, 'e')])

    w2t = params['t2_w'].T
    out = _head_call(x0, x1, x2,
                     w2t[0:D][perm, :], w2t[D:2 * D][perm, :],
                     w2t[2 * D:3 * D][perm, :], row(params['t2_b']))
    ee = jnp.concatenate([e0[:E], e1[:E]], axis=0)
    return out, ee
